# Initial kernel scaffold; baseline (speedup 1.0000x reference)
import jax
import jax.numpy as jnp
from jax.experimental import pallas as pl


def kernel(x0, x1, edge_index, W_self_0, W_neigh_0, b_0, W_self_1, W_neigh_1, b_1, W_self_2, W_neigh_2, b_2):
    raise NotImplementedError("write your pallas kernel here")


# speedup vs baseline: 4.2269x; 4.2269x over previous
import functools

import jax
import jax.numpy as jnp
from jax import lax
from jax.experimental import pallas as pl
from jax.experimental.pallas import tpu as pltpu
from jax.experimental.pallas import tpu_sc as plsc

_N = 10000
_D = 128
_NSUB = 16
_CH = 128
_BN = 1000


@functools.cache
def _sc_agg(Ec, with_deg):
    Es = Ec // _NSUB
    n_full = Es // _CH
    tail = Es - n_full * _CH
    RP = 624
    RCH = 104
    NR = RP // RCH
    RTAIL = _N - RP * _NSUB

    mesh = plsc.VectorSubcoreMesh(core_axis_name="c", subcore_axis_name="s")

    if with_deg:
        out_type = [jax.ShapeDtypeStruct((2, _N, _D), jnp.float32),
                    jax.ShapeDtypeStruct((_N,), jnp.float32),
                    jax.ShapeDtypeStruct((_N,), jnp.float32)]
    else:
        out_type = jax.ShapeDtypeStruct((2, _N, _D), jnp.float32)

    scratch = [
        pltpu.VMEM_SHARED((_N, _D), jnp.float32),
        pltpu.VMEM((_CH,), jnp.int32),
        pltpu.VMEM((_CH,), jnp.int32),
        pltpu.VMEM((tail,), jnp.int32),
        pltpu.VMEM((tail,), jnp.int32),
        pltpu.VMEM((_CH, _D), jnp.float32),
        pltpu.VMEM((tail, _D), jnp.float32),
        pltpu.VMEM((RCH, _D), jnp.float32),
        pltpu.SemaphoreType.DMA,
    ]
    if with_deg:
        scratch += [
            pltpu.VMEM_SHARED((_N,), jnp.float32),
            pltpu.VMEM((_CH,), jnp.float32),
            pltpu.VMEM((tail,), jnp.float32),
            pltpu.VMEM((1024,), jnp.float32),
        ]

    def body(t0, t1, gi0, si0, gi1, si1, out, *rest):
        if with_deg:
            deg_out0, deg_out1, *rest = rest
        (accum, gi_v, si_v, git_v, sit_v, rows_v, rowst_v, stage_v,
         sem) = rest[:9]
        if with_deg:
            deg_sh, ones_v, onest_v, dstage_v = rest[9:]

        c = lax.axis_index("c")
        s = lax.axis_index("s")
        row0 = s * RP

        def zr(r, carry):
            for k in range(_D // 16):
                stage_v[r, pl.ds(k * 16, 16)] = jnp.zeros((16,), jnp.float32)
            return carry
        lax.fori_loop(0, RCH, zr, 0)
        for r in range(NR):
            pltpu.sync_copy(stage_v, accum.at[pl.ds(row0 + r * RCH, RCH), :])

        @pl.when(s == _NSUB - 1)
        def _():
            pltpu.sync_copy(stage_v.at[pl.ds(0, RTAIL), :],
                            accum.at[pl.ds(RP * _NSUB, RTAIL), :])
        if with_deg:
            def zd(k, carry):
                dstage_v[pl.ds(k * 16, 16)] = jnp.zeros((16,), jnp.float32)
                return carry
            lax.fori_loop(0, 64, zd, 0)

            @pl.when(s < 10)
            def _():
                pltpu.sync_copy(dstage_v.at[pl.ds(0, 1000)],
                                deg_sh.at[pl.ds(s * 1000, 1000)])

            def on(k, carry):
                ones_v[pl.ds(k * 16, 16)] = jnp.full((16,), 1.0, jnp.float32)
                return carry
            lax.fori_loop(0, _CH // 16, on, 0)

            def ont(k, carry):
                onest_v[pl.ds(k * 16, 16)] = jnp.full((16,), 1.0, jnp.float32)
                return carry
            lax.fori_loop(0, tail // 16, ont, 0)
        plsc.subcore_barrier()

        ebase = s * Es

        def run(tab, gi, si):
            def chunk(j, carry):
                off = ebase + j * _CH
                pltpu.sync_copy(gi.at[pl.ds(off, _CH)], gi_v)
                pltpu.sync_copy(si.at[pl.ds(off, _CH)], si_v)
                pltpu.async_copy(tab.at[gi_v], rows_v, sem).wait()
                pltpu.sync_copy(rows_v, accum.at[si_v], add=True)
                if with_deg:
                    pltpu.sync_copy(ones_v, deg_sh.at[si_v], add=True)
                return carry
            lax.fori_loop(0, n_full, chunk, 0)
            if tail:
                off = ebase + n_full * _CH
                pltpu.sync_copy(gi.at[pl.ds(off, tail)], git_v)
                pltpu.sync_copy(si.at[pl.ds(off, tail)], sit_v)
                pltpu.async_copy(tab.at[git_v], rowst_v, sem).wait()
                pltpu.sync_copy(rowst_v, accum.at[sit_v], add=True)
                if with_deg:
                    pltpu.sync_copy(onest_v, deg_sh.at[sit_v], add=True)

        @pl.when(c == 0)
        def _():
            run(t0, gi0, si0)

        @pl.when(c == 1)
        def _():
            run(t1, gi1, si1)

        plsc.subcore_barrier()
        for r in range(NR):
            rr = row0 + r * RCH
            pltpu.sync_copy(accum.at[pl.ds(rr, RCH), :], stage_v)
            pltpu.sync_copy(stage_v, out.at[c, pl.ds(rr, RCH), :])

        @pl.when(s == _NSUB - 1)
        def _():
            rr = RP * _NSUB
            pltpu.sync_copy(accum.at[pl.ds(rr, RTAIL), :],
                            stage_v.at[pl.ds(0, RTAIL), :])
            pltpu.sync_copy(stage_v.at[pl.ds(0, RTAIL), :],
                            out.at[c, pl.ds(rr, RTAIL), :])
        if with_deg:
            @pl.when(s < 10)
            def _():
                pltpu.sync_copy(deg_sh.at[pl.ds(s * 1000, 1000)],
                                dstage_v.at[pl.ds(0, 1000)])

                @pl.when(c == 0)
                def _():
                    pltpu.sync_copy(dstage_v.at[pl.ds(0, 1000)],
                                    deg_out0.at[pl.ds(s * 1000, 1000)])

                @pl.when(c == 1)
                def _():
                    pltpu.sync_copy(dstage_v.at[pl.ds(0, 1000)],
                                    deg_out1.at[pl.ds(s * 1000, 1000)])

    return pl.kernel(body, out_type=out_type, mesh=mesh,
                     scratch_types=scratch)


def _dot(x, w):
    return jnp.dot(x, w, preferred_element_type=jnp.float32)


@functools.cache
def _tc_layer(relu_out):
    def body(a_ref, R_ref, b_ref, F_ref, ds_ref, dd_ref, Ws_ref, Wn_ref,
             bias_ref, h0_ref, h1_ref):
        inv_s = 1.0 / jnp.maximum(ds_ref[...], 1.0)
        inv_d = 1.0 / jnp.maximum(dd_ref[...], 1.0)
        Ws = Ws_ref[...]
        Wn = Wn_ref[...]
        bias = bias_ref[...]
        h0 = _dot(a_ref[...], Ws) + _dot(R_ref[0] * inv_s, Wn) + bias
        h1 = _dot(b_ref[...], Ws) + _dot(F_ref[0] * inv_d, Wn) + bias
        if relu_out:
            h0 = jnp.maximum(h0, 0.0)
            h1 = jnp.maximum(h1, 0.0)
        h0_ref[...] = h0
        h1_ref[...] = h1

    blk = lambda i: (i, 0)
    return pl.pallas_call(
        body,
        grid=(_N // _BN,),
        in_specs=[
            pl.BlockSpec((_BN, _D), blk),
            pl.BlockSpec((1, _BN, _D), lambda i: (0, i, 0)),
            pl.BlockSpec((_BN, _D), blk),
            pl.BlockSpec((1, _BN, _D), lambda i: (1, i, 0)),
            pl.BlockSpec((_BN, 1), blk),
            pl.BlockSpec((_BN, 1), blk),
            pl.BlockSpec((_D, _D), lambda i: (0, 0)),
            pl.BlockSpec((_D, _D), lambda i: (0, 0)),
            pl.BlockSpec((1, _D), lambda i: (0, 0)),
        ],
        out_specs=[pl.BlockSpec((_BN, _D), blk), pl.BlockSpec((_BN, _D), blk)],
        out_shape=[jax.ShapeDtypeStruct((_N, _D), jnp.float32)] * 2,
    )


@functools.cache
def _tc_final(Dout):
    def body(b_ref, F0_ref, F1_ref, dd_ref, Ws_ref, Wn_ref, bias_ref, o_ref):
        inv_d = 1.0 / jnp.maximum(dd_ref[...], 1.0)
        mean = (F0_ref[0] + F1_ref[0]) * inv_d
        o_ref[...] = (_dot(b_ref[...], Ws_ref[...]) + _dot(mean, Wn_ref[...])
                      + bias_ref[...])

    blk = lambda i: (i, 0)
    return pl.pallas_call(
        body,
        grid=(_N // _BN,),
        in_specs=[
            pl.BlockSpec((_BN, _D), blk),
            pl.BlockSpec((1, _BN, _D), lambda i: (0, i, 0)),
            pl.BlockSpec((1, _BN, _D), lambda i: (1, i, 0)),
            pl.BlockSpec((_BN, 1), blk),
            pl.BlockSpec((_D, Dout), lambda i: (0, 0)),
            pl.BlockSpec((_D, Dout), lambda i: (0, 0)),
            pl.BlockSpec((1, Dout), lambda i: (0, 0)),
        ],
        out_specs=pl.BlockSpec((_BN, Dout), blk),
        out_shape=jax.ShapeDtypeStruct((_N, Dout), jnp.float32),
    )


def kernel(x0, x1, edge_index, W_self_0, W_neigh_0, b_0, W_self_1, W_neigh_1,
           b_1, W_self_2, W_neigh_2, b_2):
    src = edge_index[0]
    dst = edge_index[1]
    E = src.shape[0]
    Dout = W_self_2.shape[1]

    agg0, deg0, deg1 = _sc_agg(E, True)(x1, x0, dst, src, src, dst)
    degs = deg0.reshape(_N, 1)
    degd = deg1.reshape(_N, 1)
    a1, b1 = _tc_layer(True)(x0, agg0, x1, agg0, degs, degd,
                             W_self_0, W_neigh_0, b_0.reshape(1, _D))

    agg1 = _sc_agg(E, False)(b1, a1, dst, src, src, dst)
    a2, b2 = _tc_layer(True)(a1, agg1, b1, agg1, degs, degd,
                             W_self_1, W_neigh_1, b_1.reshape(1, _D))

    h = E // 2
    agg2 = _sc_agg(h, False)(a2, a2, src[:h], dst[:h], src[h:], dst[h:])
    return _tc_final(Dout)(b2, agg2, agg2, degd,
                           W_self_2, W_neigh_2, b_2.reshape(1, Dout))


# speedup vs baseline: 7.2264x; 1.7096x over previous
import functools

import jax
import jax.numpy as jnp
from jax import lax
from jax.experimental import pallas as pl
from jax.experimental.pallas import tpu as pltpu
from jax.experimental.pallas import tpu_sc as plsc

_N = 10000
_D = 128
_NSUB = 16
_CH = 128
_NGARB = 64
_NACC = 10112
_BN = 1000


@functools.cache
def _sc_agg(rows_total, with_deg):
    RP = rows_total // _NSUB
    IBLK = 16
    assert RP % IBLK == 0
    NBLK = RP // IBLK

    mesh = plsc.VectorSubcoreMesh(core_axis_name="c", subcore_axis_name="s")

    if with_deg:
        out_type = [jax.ShapeDtypeStruct((2, _N, _D), jnp.float32),
                    jax.ShapeDtypeStruct((_N,), jnp.float32),
                    jax.ShapeDtypeStruct((_N,), jnp.float32)]
    else:
        out_type = jax.ShapeDtypeStruct((2, _N, _D), jnp.float32)

    scratch = [
        pltpu.VMEM_SHARED((_NACC, _D), jnp.float32),
        pltpu.VMEM((2 * IBLK, _CH), jnp.int32),
        pltpu.VMEM((2 * IBLK, _CH), jnp.int32),
        pltpu.VMEM((_CH, _D), jnp.float32),
        pltpu.VMEM((_CH, _D), jnp.float32),
        pltpu.SemaphoreType.DMA,
        pltpu.SemaphoreType.DMA,
        pltpu.SemaphoreType.DMA,
        pltpu.SemaphoreType.DMA,
        pltpu.SemaphoreType.DMA,
    ]
    if with_deg:
        scratch += [
            pltpu.VMEM_SHARED((_NACC,), jnp.float32),
            pltpu.VMEM((_CH,), jnp.float32),
            pltpu.VMEM((1024,), jnp.float32),
            pltpu.SemaphoreType.DMA,
        ]

    ZP = _NACC // _NSUB
    ZCH = [128, 128, 128, 128, 120]
    OP = 624
    OCH = [128, 128, 128, 128, 112]
    RTAIL = _N - OP * _NSUB

    def body(t0, t1, gi0, si0, gi1, si1, out, *rest):
        if with_deg:
            deg_out0, deg_out1, *rest = rest
        accum, gi_v, si_v, r0buf, r1buf, isem, gsem0, gsem1, ssem0, ssem1 = (
            rest[:10])
        rows = (r0buf, r1buf)
        gsem = (gsem0, gsem1)
        ssem = (ssem0, ssem1)
        if with_deg:
            deg_sh, ones_v, dstage_v, dsem = rest[10:]

        c = lax.axis_index("c")
        s = lax.axis_index("s")
        r0 = s * RP

        def idx_fetch(gi, si, blk):
            half = lax.rem(blk, 2) * IBLK
            pltpu.async_copy(gi.at[pl.ds(r0 + blk * IBLK, IBLK), :],
                             gi_v.at[pl.ds(half, IBLK), :], isem)
            pltpu.async_copy(si.at[pl.ds(r0 + blk * IBLK, IBLK), :],
                             si_v.at[pl.ds(half, IBLK), :], isem)

        def idx_wait():
            pltpu.make_async_copy(gi0.at[pl.ds(0, IBLK), :],
                                  gi_v.at[pl.ds(0, IBLK), :], isem).wait()
            pltpu.make_async_copy(si0.at[pl.ds(0, IBLK), :],
                                  si_v.at[pl.ds(0, IBLK), :], isem).wait()

        @pl.when(c == 0)
        def _():
            idx_fetch(gi0, si0, 0)

        @pl.when(c == 1)
        def _():
            idx_fetch(gi1, si1, 0)

        def zrow(r, carry):
            for k in range(_D // 16):
                r0buf[r, pl.ds(k * 16, 16)] = jnp.zeros((16,), jnp.float32)
            return carry
        lax.fori_loop(0, _CH, zrow, 0)
        zoff = 0
        for sz in ZCH:
            pltpu.async_copy(r0buf.at[pl.ds(0, sz), :],
                             accum.at[pl.ds(s * ZP + zoff, sz), :], ssem0)
            zoff += sz
        if with_deg:
            def zd(k, carry):
                dstage_v[pl.ds(k * 16, 16)] = jnp.zeros((16,), jnp.float32)
                return carry
            lax.fori_loop(0, 64, zd, 0)

            @pl.when(s < 9)
            def _():
                pltpu.async_copy(dstage_v, deg_sh.at[pl.ds(s * 1024, 1024)],
                                 dsem)

            @pl.when(s == 9)
            def _():
                pltpu.async_copy(dstage_v.at[pl.ds(0, 896)],
                                 deg_sh.at[pl.ds(9216, 896)], dsem)

            def on(k, carry):
                ones_v[pl.ds(k * 16, 16)] = jnp.full((16,), 1.0, jnp.float32)
                return carry
            lax.fori_loop(0, _CH // 16, on, 0)
        zoff = 0
        for sz in ZCH:
            pltpu.make_async_copy(r0buf.at[pl.ds(0, sz), :],
                                  accum.at[pl.ds(s * ZP + zoff, sz), :],
                                  ssem0).wait()
            zoff += sz
        if with_deg:
            @pl.when(s < 9)
            def _():
                pltpu.make_async_copy(dstage_v,
                                      deg_sh.at[pl.ds(s * 1024, 1024)],
                                      dsem).wait()

            @pl.when(s == 9)
            def _():
                pltpu.make_async_copy(dstage_v.at[pl.ds(0, 896)],
                                      deg_sh.at[pl.ds(9216, 896)],
                                      dsem).wait()
        idx_wait()
        plsc.subcore_barrier()

        def run(tab, gi, si):
            def gather(j, b):
                pltpu.async_copy(tab.at[gi_v.at[lax.rem(j, 2 * IBLK)]],
                                 rows[b], gsem[b])

            def scatter(j, b):
                jm = lax.rem(j, 2 * IBLK)
                pltpu.async_copy(rows[b], accum.at[si_v.at[jm]], ssem[b],
                                 add=True)
                if with_deg:
                    pltpu.async_copy(ones_v, deg_sh.at[si_v.at[jm]], dsem,
                                     add=True)

            def gwait(b):
                pltpu.make_async_copy(tab.at[pl.ds(0, _CH), :], rows[b],
                                      gsem[b]).wait()

            def swait(b):
                pltpu.make_async_copy(tab.at[pl.ds(0, _CH), :], rows[b],
                                      ssem[b]).wait()

            gather(0, 0)
            gather(1, 1)

            def block(g, carry):
                idx_fetch(gi, si, g + 1)
                base = g * IBLK
                for grp in range(IBLK // 2):
                    for b in range(2):
                        gwait(b)
                        scatter(base + grp * 2 + b, b)
                    if grp == IBLK // 2 - 1:
                        idx_wait()
                    for b in range(2):
                        swait(b)
                        gather(base + grp * 2 + b + 2, b)
                return carry
            lax.fori_loop(0, NBLK - 1, block, 0)

            base = (NBLK - 1) * IBLK
            for grp in range(IBLK // 2):
                for b in range(2):
                    gwait(b)
                    scatter(base + grp * 2 + b, b)
                if grp < IBLK // 2 - 1:
                    for b in range(2):
                        swait(b)
                        gather(base + grp * 2 + b + 2, b)
            swait(0)
            swait(1)
            if with_deg:
                def drain(j, carry):
                    pltpu.make_async_copy(ones_v, deg_sh.at[si_v.at[0]],
                                          dsem).wait()
                    return carry
                lax.fori_loop(0, RP, drain, 0)

        @pl.when(c == 0)
        def _():
            run(t0, gi0, si0)

        @pl.when(c == 1)
        def _():
            run(t1, gi1, si1)

        plsc.subcore_barrier()
        row0 = s * OP
        ooff = 0
        for k, sz in enumerate(OCH):
            b = k % 2
            rr = row0 + ooff
            pltpu.sync_copy(accum.at[pl.ds(rr, sz), :],
                            rows[b].at[pl.ds(0, sz), :])
            pltpu.async_copy(rows[b].at[pl.ds(0, sz), :],
                             out.at[c, pl.ds(rr, sz), :], ssem[b])
            ooff += sz
        ooff = 0
        for k, sz in enumerate(OCH):
            b = k % 2
            rr = row0 + ooff
            pltpu.make_async_copy(rows[b].at[pl.ds(0, sz), :],
                                  out.at[c, pl.ds(rr, sz), :], ssem[b]).wait()
            ooff += sz

        @pl.when(s == _NSUB - 1)
        def _():
            rr = OP * _NSUB
            pltpu.sync_copy(accum.at[pl.ds(rr, RTAIL), :],
                            r0buf.at[pl.ds(0, RTAIL), :])
            pltpu.sync_copy(r0buf.at[pl.ds(0, RTAIL), :],
                            out.at[c, pl.ds(rr, RTAIL), :])
        if with_deg:
            @pl.when(s < 10)
            def _():
                pltpu.sync_copy(deg_sh.at[pl.ds(s * 1000, 1000)],
                                dstage_v.at[pl.ds(0, 1000)])

                @pl.when(c == 0)
                def _():
                    pltpu.sync_copy(dstage_v.at[pl.ds(0, 1000)],
                                    deg_out0.at[pl.ds(s * 1000, 1000)])

                @pl.when(c == 1)
                def _():
                    pltpu.sync_copy(dstage_v.at[pl.ds(0, 1000)],
                                    deg_out1.at[pl.ds(s * 1000, 1000)])

    return pl.kernel(body, out_type=out_type, mesh=mesh,
                     scratch_types=scratch)


def _pad_idx(idx, rows, scatter):
    pad = rows * _CH - idx.shape[0]
    fill = jax.lax.iota(jnp.int32, pad)
    fill = _N + (fill % _NGARB) if scatter else fill % _N
    return jnp.concatenate([idx, fill]).reshape(rows, _CH)


def _dot(x, w):
    return jnp.dot(x, w, preferred_element_type=jnp.float32)


@functools.cache
def _tc_layer(relu_out):
    def body(a_ref, R_ref, b_ref, F_ref, ds_ref, dd_ref, Ws_ref, Wn_ref,
             bias_ref, h0_ref, h1_ref):
        inv_s = 1.0 / jnp.maximum(ds_ref[...], 1.0)
        inv_d = 1.0 / jnp.maximum(dd_ref[...], 1.0)
        Ws = Ws_ref[...]
        Wn = Wn_ref[...]
        bias = bias_ref[...]
        h0 = _dot(a_ref[...], Ws) + _dot(R_ref[0] * inv_s, Wn) + bias
        h1 = _dot(b_ref[...], Ws) + _dot(F_ref[0] * inv_d, Wn) + bias
        if relu_out:
            h0 = jnp.maximum(h0, 0.0)
            h1 = jnp.maximum(h1, 0.0)
        h0_ref[...] = h0
        h1_ref[...] = h1

    blk = lambda i: (i, 0)
    return pl.pallas_call(
        body,
        grid=(_N // _BN,),
        in_specs=[
            pl.BlockSpec((_BN, _D), blk),
            pl.BlockSpec((1, _BN, _D), lambda i: (0, i, 0)),
            pl.BlockSpec((_BN, _D), blk),
            pl.BlockSpec((1, _BN, _D), lambda i: (1, i, 0)),
            pl.BlockSpec((_BN, 1), blk),
            pl.BlockSpec((_BN, 1), blk),
            pl.BlockSpec((_D, _D), lambda i: (0, 0)),
            pl.BlockSpec((_D, _D), lambda i: (0, 0)),
            pl.BlockSpec((1, _D), lambda i: (0, 0)),
        ],
        out_specs=[pl.BlockSpec((_BN, _D), blk), pl.BlockSpec((_BN, _D), blk)],
        out_shape=[jax.ShapeDtypeStruct((_N, _D), jnp.float32)] * 2,
    )


@functools.cache
def _tc_final(Dout):
    def body(b_ref, F0_ref, F1_ref, dd_ref, Ws_ref, Wn_ref, bias_ref, o_ref):
        inv_d = 1.0 / jnp.maximum(dd_ref[...], 1.0)
        mean = (F0_ref[0] + F1_ref[0]) * inv_d
        o_ref[...] = (_dot(b_ref[...], Ws_ref[...]) + _dot(mean, Wn_ref[...])
                      + bias_ref[...])

    blk = lambda i: (i, 0)
    return pl.pallas_call(
        body,
        grid=(_N // _BN,),
        in_specs=[
            pl.BlockSpec((_BN, _D), blk),
            pl.BlockSpec((1, _BN, _D), lambda i: (0, i, 0)),
            pl.BlockSpec((1, _BN, _D), lambda i: (1, i, 0)),
            pl.BlockSpec((_BN, 1), blk),
            pl.BlockSpec((_D, Dout), lambda i: (0, 0)),
            pl.BlockSpec((_D, Dout), lambda i: (0, 0)),
            pl.BlockSpec((1, Dout), lambda i: (0, 0)),
        ],
        out_specs=pl.BlockSpec((_BN, Dout), blk),
        out_shape=jax.ShapeDtypeStruct((_N, Dout), jnp.float32),
    )


def kernel(x0, x1, edge_index, W_self_0, W_neigh_0, b_0, W_self_1, W_neigh_1,
           b_1, W_self_2, W_neigh_2, b_2):
    src = edge_index[0]
    dst = edge_index[1]
    E = src.shape[0]
    Dout = W_self_2.shape[1]

    unit = _CH * _NSUB * 16
    rows_a = -(-E // unit) * _NSUB * 16
    h = E // 2
    rows_h = -(-h // unit) * _NSUB * 16
    src_g = _pad_idx(src, rows_a, False)
    src_s = _pad_idx(src, rows_a, True)
    dst_g = _pad_idx(dst, rows_a, False)
    dst_s = _pad_idx(dst, rows_a, True)

    agg0, deg0, deg1 = _sc_agg(rows_a, True)(x1, x0, dst_g, src_s, src_g,
                                             dst_s)
    degs = deg0.reshape(_N, 1)
    degd = deg1.reshape(_N, 1)
    a1, b1 = _tc_layer(True)(x0, agg0, x1, agg0, degs, degd,
                             W_self_0, W_neigh_0, b_0.reshape(1, _D))

    agg1 = _sc_agg(rows_a, False)(b1, a1, dst_g, src_s, src_g, dst_s)
    a2, b2 = _tc_layer(True)(a1, agg1, b1, agg1, degs, degd,
                             W_self_1, W_neigh_1, b_1.reshape(1, _D))

    agg2 = _sc_agg(rows_h, False)(
        a2, a2,
        _pad_idx(src[:h], rows_h, False), _pad_idx(dst[:h], rows_h, True),
        _pad_idx(src[h:], rows_h, False), _pad_idx(dst[h:], rows_h, True))
    return _tc_final(Dout)(b2, agg2, agg2, degd,
                           W_self_2, W_neigh_2, b_2.reshape(1, Dout))


# speedup vs baseline: 7.6921x; 1.0644x over previous
import functools

import jax
import jax.numpy as jnp
from jax import lax
from jax.experimental import pallas as pl
from jax.experimental.pallas import tpu as pltpu
from jax.experimental.pallas import tpu_sc as plsc

_N = 10000
_D = 128
_NSUB = 16
_NGARB = 64
_NACC = 10112
_BN = 1000


@functools.cache
def _sc_agg(rows_total, with_deg, ch, nbuf, iblk):
    RP = rows_total // _NSUB
    assert RP % iblk == 0 and iblk % nbuf == 0 and iblk % 8 == 0
    NBLK = RP // iblk
    NGRP = iblk // nbuf

    mesh = plsc.VectorSubcoreMesh(core_axis_name="c", subcore_axis_name="s")

    if with_deg:
        out_type = [jax.ShapeDtypeStruct((2, _N, _D), jnp.float32),
                    jax.ShapeDtypeStruct((_N,), jnp.float32),
                    jax.ShapeDtypeStruct((_N,), jnp.float32)]
    else:
        out_type = jax.ShapeDtypeStruct((2, _N, _D), jnp.float32)

    scratch = [
        pltpu.VMEM_SHARED((_NACC, _D), jnp.float32),
        pltpu.VMEM((2 * iblk, ch), jnp.int32),
        pltpu.VMEM((2 * iblk, ch), jnp.int32),
    ]
    scratch += [pltpu.VMEM((ch, _D), jnp.float32) for _ in range(nbuf)]
    scratch += [pltpu.SemaphoreType.DMA]
    scratch += [pltpu.SemaphoreType.DMA for _ in range(nbuf)]
    scratch += [pltpu.SemaphoreType.DMA for _ in range(nbuf)]
    if with_deg:
        scratch += [
            pltpu.VMEM_SHARED((_NACC,), jnp.float32),
            pltpu.VMEM((-(-ch // 16) * 16,), jnp.float32),
            pltpu.VMEM((640,), jnp.float32),
            pltpu.SemaphoreType.DMA,
        ]

    ZP = _NACC // _NSUB
    nz, zr_ = divmod(ZP, ch)
    ZCH = [ch] * nz + ([zr_] if zr_ else [])
    OP = 624
    no, or_ = divmod(OP, ch)
    OCH = [ch] * no + ([or_] if or_ else [])
    RTAIL = _N - OP * _NSUB
    assert all(x % 8 == 0 for x in ZCH + OCH)

    def body(t0, t1, gi0, si0, gi1, si1, out, *rest):
        if with_deg:
            deg_out0, deg_out1, *rest = rest
        accum, gi_v, si_v = rest[:3]
        rows = rest[3:3 + nbuf]
        isem = rest[3 + nbuf]
        gsem = rest[4 + nbuf:4 + 2 * nbuf]
        ssem = rest[4 + 2 * nbuf:4 + 3 * nbuf]
        if with_deg:
            deg_sh, ones_v, dstage_v, dsem = rest[4 + 3 * nbuf:]
        r0buf = rows[0]

        c = lax.axis_index("c")
        s = lax.axis_index("s")
        r0 = s * RP

        def idx_fetch(gi, si, blk):
            half = lax.rem(blk, 2) * iblk
            pltpu.async_copy(gi.at[pl.ds(r0 + blk * iblk, iblk), :],
                             gi_v.at[pl.ds(half, iblk), :], isem)
            pltpu.async_copy(si.at[pl.ds(r0 + blk * iblk, iblk), :],
                             si_v.at[pl.ds(half, iblk), :], isem)

        def idx_wait():
            pltpu.make_async_copy(gi0.at[pl.ds(0, iblk), :],
                                  gi_v.at[pl.ds(0, iblk), :], isem).wait()
            pltpu.make_async_copy(si0.at[pl.ds(0, iblk), :],
                                  si_v.at[pl.ds(0, iblk), :], isem).wait()

        @pl.when(c == 0)
        def _():
            idx_fetch(gi0, si0, 0)

        @pl.when(c == 1)
        def _():
            idx_fetch(gi1, si1, 0)

        def zrow(r, carry):
            for k in range(_D // 16):
                r0buf[r, pl.ds(k * 16, 16)] = jnp.zeros((16,), jnp.float32)
            return carry
        lax.fori_loop(0, ch, zrow, 0)
        zoff = 0
        for sz in ZCH:
            pltpu.async_copy(r0buf.at[pl.ds(0, sz), :],
                             accum.at[pl.ds(s * ZP + zoff, sz), :], ssem[0])
            zoff += sz
        if with_deg:
            def zd(k, carry):
                dstage_v[pl.ds(k * 16, 16)] = jnp.zeros((16,), jnp.float32)
                return carry
            lax.fori_loop(0, 40, zd, 0)

            @pl.when(s < 15)
            def _():
                pltpu.async_copy(dstage_v, deg_sh.at[pl.ds(s * 640, 640)],
                                 dsem)

            @pl.when(s == 15)
            def _():
                pltpu.async_copy(dstage_v.at[pl.ds(0, 512)],
                                 deg_sh.at[pl.ds(9600, 512)], dsem)

            def on(k, carry):
                ones_v[pl.ds(k * 16, 16)] = jnp.full((16,), 1.0, jnp.float32)
                return carry
            lax.fori_loop(0, -(-ch // 16), on, 0)
        zoff = 0
        for sz in ZCH:
            pltpu.make_async_copy(r0buf.at[pl.ds(0, sz), :],
                                  accum.at[pl.ds(s * ZP + zoff, sz), :],
                                  ssem[0]).wait()
            zoff += sz
        if with_deg:
            @pl.when(s < 15)
            def _():
                pltpu.make_async_copy(dstage_v,
                                      deg_sh.at[pl.ds(s * 640, 640)],
                                      dsem).wait()

            @pl.when(s == 15)
            def _():
                pltpu.make_async_copy(dstage_v.at[pl.ds(0, 512)],
                                      deg_sh.at[pl.ds(9600, 512)],
                                      dsem).wait()
        idx_wait()
        plsc.subcore_barrier()

        def run(tab, gi, si):
            def gather(j, b):
                pltpu.async_copy(tab.at[gi_v.at[lax.rem(j, 2 * iblk)]],
                                 rows[b], gsem[b])

            def scatter(j, b):
                jm = lax.rem(j, 2 * iblk)
                pltpu.async_copy(rows[b], accum.at[si_v.at[jm]], ssem[b],
                                 add=True)
                if with_deg:
                    pltpu.async_copy(ones_v.at[pl.ds(0, ch)],
                                     deg_sh.at[si_v.at[jm]], dsem, add=True)

            def gwait(b):
                pltpu.make_async_copy(tab.at[pl.ds(0, ch), :], rows[b],
                                      gsem[b]).wait()

            def swait(b):
                pltpu.make_async_copy(tab.at[pl.ds(0, ch), :], rows[b],
                                      ssem[b]).wait()

            for b in range(nbuf):
                gather(b, b)

            def block(g, carry):
                idx_fetch(gi, si, g + 1)
                base = g * iblk
                for grp in range(NGRP):
                    for b in range(nbuf):
                        gwait(b)
                        scatter(base + grp * nbuf + b, b)
                    if grp == NGRP - 1:
                        idx_wait()
                    for b in range(nbuf):
                        swait(b)
                        gather(base + grp * nbuf + b + nbuf, b)
                return carry
            lax.fori_loop(0, NBLK - 1, block, 0)

            base = (NBLK - 1) * iblk
            for grp in range(NGRP):
                for b in range(nbuf):
                    gwait(b)
                    scatter(base + grp * nbuf + b, b)
                if grp < NGRP - 1:
                    for b in range(nbuf):
                        swait(b)
                        gather(base + grp * nbuf + b + nbuf, b)
            for b in range(nbuf):
                swait(b)
            if with_deg:
                def drain(j, carry):
                    pltpu.make_async_copy(ones_v.at[pl.ds(0, ch)],
                                          deg_sh.at[si_v.at[0]],
                                          dsem).wait()
                    return carry
                lax.fori_loop(0, RP, drain, 0)

        @pl.when(c == 0)
        def _():
            run(t0, gi0, si0)

        @pl.when(c == 1)
        def _():
            run(t1, gi1, si1)

        plsc.subcore_barrier()
        row0 = s * OP
        ooff = 0
        for k, sz in enumerate(OCH):
            b = k % nbuf
            rr = row0 + ooff
            pltpu.sync_copy(accum.at[pl.ds(rr, sz), :],
                            rows[b].at[pl.ds(0, sz), :])
            pltpu.async_copy(rows[b].at[pl.ds(0, sz), :],
                             out.at[c, pl.ds(rr, sz), :], ssem[b])
            ooff += sz
        ooff = 0
        for k, sz in enumerate(OCH):
            b = k % nbuf
            rr = row0 + ooff
            pltpu.make_async_copy(rows[b].at[pl.ds(0, sz), :],
                                  out.at[c, pl.ds(rr, sz), :], ssem[b]).wait()
            ooff += sz

        @pl.when(s == _NSUB - 1)
        def _():
            rr = OP * _NSUB
            pltpu.sync_copy(accum.at[pl.ds(rr, RTAIL), :],
                            r0buf.at[pl.ds(0, RTAIL), :])
            pltpu.sync_copy(r0buf.at[pl.ds(0, RTAIL), :],
                            out.at[c, pl.ds(rr, RTAIL), :])
        if with_deg:
            @pl.when(s < 15)
            def _():
                pltpu.sync_copy(deg_sh.at[pl.ds(s * 640, 640)], dstage_v)

                @pl.when(c == 0)
                def _():
                    pltpu.sync_copy(dstage_v,
                                    deg_out0.at[pl.ds(s * 640, 640)])

                @pl.when(c == 1)
                def _():
                    pltpu.sync_copy(dstage_v,
                                    deg_out1.at[pl.ds(s * 640, 640)])

            @pl.when(s == 15)
            def _():
                pltpu.sync_copy(deg_sh.at[pl.ds(9600, 400)],
                                dstage_v.at[pl.ds(0, 400)])

                @pl.when(c == 0)
                def _():
                    pltpu.sync_copy(dstage_v.at[pl.ds(0, 400)],
                                    deg_out0.at[pl.ds(9600, 400)])

                @pl.when(c == 1)
                def _():
                    pltpu.sync_copy(dstage_v.at[pl.ds(0, 400)],
                                    deg_out1.at[pl.ds(9600, 400)])

    return pl.kernel(body, out_type=out_type, mesh=mesh,
                     scratch_types=scratch)


def _pad_rows(n_edges, ch, iblk):
    unit = ch * _NSUB * iblk
    return -(-n_edges // unit) * _NSUB * iblk


def _pad_idx(idx, rows, ch, scatter):
    pad = rows * ch - idx.shape[0]
    fill = jax.lax.iota(jnp.int32, pad)
    fill = _N + (fill % _NGARB) if scatter else fill % _N
    return jnp.concatenate([idx, fill]).reshape(rows, ch)


def _dot(x, w):
    return jnp.dot(x, w, preferred_element_type=jnp.float32)


@functools.cache
def _tc_layer(relu_out):
    def body(a_ref, R_ref, b_ref, F_ref, ds_ref, dd_ref, Ws_ref, Wn_ref,
             bias_ref, h0_ref, h1_ref):
        inv_s = 1.0 / jnp.maximum(ds_ref[...], 1.0)
        inv_d = 1.0 / jnp.maximum(dd_ref[...], 1.0)
        Ws = Ws_ref[...]
        Wn = Wn_ref[...]
        bias = bias_ref[...]
        h0 = _dot(a_ref[...], Ws) + _dot(R_ref[0] * inv_s, Wn) + bias
        h1 = _dot(b_ref[...], Ws) + _dot(F_ref[0] * inv_d, Wn) + bias
        if relu_out:
            h0 = jnp.maximum(h0, 0.0)
            h1 = jnp.maximum(h1, 0.0)
        h0_ref[...] = h0
        h1_ref[...] = h1

    blk = lambda i: (i, 0)
    return pl.pallas_call(
        body,
        grid=(_N // _BN,),
        in_specs=[
            pl.BlockSpec((_BN, _D), blk),
            pl.BlockSpec((1, _BN, _D), lambda i: (0, i, 0)),
            pl.BlockSpec((_BN, _D), blk),
            pl.BlockSpec((1, _BN, _D), lambda i: (1, i, 0)),
            pl.BlockSpec((_BN, 1), blk),
            pl.BlockSpec((_BN, 1), blk),
            pl.BlockSpec((_D, _D), lambda i: (0, 0)),
            pl.BlockSpec((_D, _D), lambda i: (0, 0)),
            pl.BlockSpec((1, _D), lambda i: (0, 0)),
        ],
        out_specs=[pl.BlockSpec((_BN, _D), blk), pl.BlockSpec((_BN, _D), blk)],
        out_shape=[jax.ShapeDtypeStruct((_N, _D), jnp.float32)] * 2,
    )


@functools.cache
def _tc_final(Dout):
    def body(b_ref, F0_ref, F1_ref, dd_ref, Ws_ref, Wn_ref, bias_ref, o_ref):
        inv_d = 1.0 / jnp.maximum(dd_ref[...], 1.0)
        mean = (F0_ref[0] + F1_ref[0]) * inv_d
        o_ref[...] = (_dot(b_ref[...], Ws_ref[...]) + _dot(mean, Wn_ref[...])
                      + bias_ref[...])

    blk = lambda i: (i, 0)
    return pl.pallas_call(
        body,
        grid=(_N // _BN,),
        in_specs=[
            pl.BlockSpec((_BN, _D), blk),
            pl.BlockSpec((1, _BN, _D), lambda i: (0, i, 0)),
            pl.BlockSpec((1, _BN, _D), lambda i: (1, i, 0)),
            pl.BlockSpec((_BN, 1), blk),
            pl.BlockSpec((_D, Dout), lambda i: (0, 0)),
            pl.BlockSpec((_D, Dout), lambda i: (0, 0)),
            pl.BlockSpec((1, Dout), lambda i: (0, 0)),
        ],
        out_specs=pl.BlockSpec((_BN, Dout), blk),
        out_shape=jax.ShapeDtypeStruct((_N, Dout), jnp.float32),
    )


_CHA, _NBA, _IBA = 88, 3, 24
_CHB, _NBB, _IBB = 128, 2, 16


def kernel(x0, x1, edge_index, W_self_0, W_neigh_0, b_0, W_self_1, W_neigh_1,
           b_1, W_self_2, W_neigh_2, b_2):
    src = edge_index[0]
    dst = edge_index[1]
    E = src.shape[0]
    Dout = W_self_2.shape[1]

    rows_a = _pad_rows(E, _CHA, _IBA)
    h = E // 2
    rows_h = _pad_rows(h, _CHB, _IBB)
    src_g = _pad_idx(src, rows_a, _CHA, False)
    src_s = _pad_idx(src, rows_a, _CHA, True)
    dst_g = _pad_idx(dst, rows_a, _CHA, False)
    dst_s = _pad_idx(dst, rows_a, _CHA, True)

    agg0, deg0, deg1 = _sc_agg(rows_a, True, _CHA, _NBA, _IBA)(
        x1, x0, dst_g, src_s, src_g, dst_s)
    degs = deg0.reshape(_N, 1)
    degd = deg1.reshape(_N, 1)
    a1, b1 = _tc_layer(True)(x0, agg0, x1, agg0, degs, degd,
                             W_self_0, W_neigh_0, b_0.reshape(1, _D))

    agg1 = _sc_agg(rows_a, False, _CHA, _NBA, _IBA)(
        b1, a1, dst_g, src_s, src_g, dst_s)
    a2, b2 = _tc_layer(True)(a1, agg1, b1, agg1, degs, degd,
                             W_self_1, W_neigh_1, b_1.reshape(1, _D))

    agg2 = _sc_agg(rows_h, False, _CHB, _NBB, _IBB)(
        a2, a2,
        _pad_idx(src[:h], rows_h, _CHB, False),
        _pad_idx(dst[:h], rows_h, _CHB, True),
        _pad_idx(src[h:], rows_h, _CHB, False),
        _pad_idx(dst[h:], rows_h, _CHB, True))
    return _tc_final(Dout)(b2, agg2, agg2, degd,
                           W_self_2, W_neigh_2, b_2.reshape(1, Dout))


# speedup vs baseline: 7.9087x; 1.0282x over previous
import functools

import jax
import jax.numpy as jnp
from jax import lax
from jax.experimental import pallas as pl
from jax.experimental.pallas import tpu as pltpu
from jax.experimental.pallas import tpu_sc as plsc

_N = 10000
_D = 128
_NSUB = 16
_NGARB = 64
_NACC = 10112
_BN = 1000


@functools.cache
def _sc_agg(rows_total, with_deg, ch, nbuf, iblk):
    RP = rows_total // _NSUB
    assert RP % iblk == 0 and iblk % nbuf == 0 and iblk % 8 == 0
    NBLK = RP // iblk
    NGRP = iblk // nbuf

    mesh = plsc.VectorSubcoreMesh(core_axis_name="c", subcore_axis_name="s")

    if with_deg:
        out_type = [jax.ShapeDtypeStruct((2, _N, _D), jnp.float32),
                    jax.ShapeDtypeStruct((_N,), jnp.float32),
                    jax.ShapeDtypeStruct((_N,), jnp.float32)]
    else:
        out_type = jax.ShapeDtypeStruct((2, _N, _D), jnp.float32)

    scratch = [
        pltpu.VMEM_SHARED((_NACC, _D), jnp.float32),
        pltpu.VMEM((2 * iblk, ch), jnp.int32),
        pltpu.VMEM((2 * iblk, ch), jnp.int32),
    ]
    scratch += [pltpu.VMEM((ch, _D), jnp.float32) for _ in range(nbuf)]
    scratch += [pltpu.SemaphoreType.DMA]
    scratch += [pltpu.SemaphoreType.DMA for _ in range(nbuf)]
    scratch += [pltpu.SemaphoreType.DMA for _ in range(nbuf)]
    if with_deg:
        scratch += [
            pltpu.VMEM_SHARED((_NACC,), jnp.float32),
            pltpu.VMEM((-(-ch // 16) * 16,), jnp.float32),
            pltpu.VMEM((640,), jnp.float32),
            pltpu.SemaphoreType.DMA,
        ]

    ZP = _NACC // _NSUB
    nz, zr_ = divmod(ZP, ch)
    ZCH = [ch] * nz + ([zr_] if zr_ else [])
    OP = 624
    no, or_ = divmod(OP, ch)
    OCH = [ch] * no + ([or_] if or_ else [])
    RTAIL = _N - OP * _NSUB
    assert all(x % 8 == 0 for x in ZCH + OCH)

    def body(t0, t1, gi0, si0, gi1, si1, out, *rest):
        if with_deg:
            deg_out0, deg_out1, *rest = rest
        accum, gi_v, si_v = rest[:3]
        rows = rest[3:3 + nbuf]
        isem = rest[3 + nbuf]
        gsem = rest[4 + nbuf:4 + 2 * nbuf]
        ssem = rest[4 + 2 * nbuf:4 + 3 * nbuf]
        if with_deg:
            deg_sh, ones_v, dstage_v, dsem = rest[4 + 3 * nbuf:]
        r0buf = rows[0]

        c = lax.axis_index("c")
        s = lax.axis_index("s")
        r0 = s * RP

        def idx_fetch(gi, si, blk):
            half = lax.rem(blk, 2) * iblk
            pltpu.async_copy(gi.at[pl.ds(r0 + blk * iblk, iblk), :],
                             gi_v.at[pl.ds(half, iblk), :], isem)
            pltpu.async_copy(si.at[pl.ds(r0 + blk * iblk, iblk), :],
                             si_v.at[pl.ds(half, iblk), :], isem)

        def idx_wait():
            pltpu.make_async_copy(gi0.at[pl.ds(0, iblk), :],
                                  gi_v.at[pl.ds(0, iblk), :], isem).wait()
            pltpu.make_async_copy(si0.at[pl.ds(0, iblk), :],
                                  si_v.at[pl.ds(0, iblk), :], isem).wait()

        @pl.when(c == 0)
        def _():
            idx_fetch(gi0, si0, 0)

        @pl.when(c == 1)
        def _():
            idx_fetch(gi1, si1, 0)

        def zrow(r, carry):
            for k in range(_D // 16):
                r0buf[r, pl.ds(k * 16, 16)] = jnp.zeros((16,), jnp.float32)
            return carry
        lax.fori_loop(0, ch, zrow, 0)
        zoff = 0
        for sz in ZCH:
            pltpu.async_copy(r0buf.at[pl.ds(0, sz), :],
                             accum.at[pl.ds(s * ZP + zoff, sz), :], ssem[0])
            zoff += sz
        if with_deg:
            def zd(k, carry):
                dstage_v[pl.ds(k * 16, 16)] = jnp.zeros((16,), jnp.float32)
                return carry
            lax.fori_loop(0, 40, zd, 0)

            @pl.when(s < 15)
            def _():
                pltpu.async_copy(dstage_v, deg_sh.at[pl.ds(s * 640, 640)],
                                 dsem)

            @pl.when(s == 15)
            def _():
                pltpu.async_copy(dstage_v.at[pl.ds(0, 512)],
                                 deg_sh.at[pl.ds(9600, 512)], dsem)

            def on(k, carry):
                ones_v[pl.ds(k * 16, 16)] = jnp.full((16,), 1.0, jnp.float32)
                return carry
            lax.fori_loop(0, -(-ch // 16), on, 0)
        zoff = 0
        for sz in ZCH:
            pltpu.make_async_copy(r0buf.at[pl.ds(0, sz), :],
                                  accum.at[pl.ds(s * ZP + zoff, sz), :],
                                  ssem[0]).wait()
            zoff += sz
        if with_deg:
            @pl.when(s < 15)
            def _():
                pltpu.make_async_copy(dstage_v,
                                      deg_sh.at[pl.ds(s * 640, 640)],
                                      dsem).wait()

            @pl.when(s == 15)
            def _():
                pltpu.make_async_copy(dstage_v.at[pl.ds(0, 512)],
                                      deg_sh.at[pl.ds(9600, 512)],
                                      dsem).wait()
        idx_wait()
        plsc.subcore_barrier()

        def run(tab, gi, si):
            def gather(j, b):
                pltpu.async_copy(tab.at[gi_v.at[lax.rem(j, 2 * iblk)]],
                                 rows[b], gsem[b])

            def scatter(j, b):
                jm = lax.rem(j, 2 * iblk)
                pltpu.async_copy(rows[b], accum.at[si_v.at[jm]], ssem[b],
                                 add=True)
                if with_deg:
                    pltpu.async_copy(ones_v.at[pl.ds(0, ch)],
                                     deg_sh.at[si_v.at[jm]], dsem, add=True)

            def gwait(b):
                pltpu.make_async_copy(tab.at[pl.ds(0, ch), :], rows[b],
                                      gsem[b]).wait()

            def swait(b):
                pltpu.make_async_copy(tab.at[pl.ds(0, ch), :], rows[b],
                                      ssem[b]).wait()

            for b in range(nbuf):
                gather(b, b)

            def block(g, carry):
                idx_fetch(gi, si, g + 1)
                base = g * iblk
                for grp in range(NGRP):
                    for b in range(nbuf):
                        gwait(b)
                        scatter(base + grp * nbuf + b, b)
                    if grp == NGRP - 1:
                        idx_wait()
                    for b in range(nbuf):
                        swait(b)
                        gather(base + grp * nbuf + b + nbuf, b)
                return carry
            lax.fori_loop(0, NBLK - 1, block, 0)

            base = (NBLK - 1) * iblk
            for grp in range(NGRP):
                for b in range(nbuf):
                    gwait(b)
                    scatter(base + grp * nbuf + b, b)
                if grp < NGRP - 1:
                    for b in range(nbuf):
                        swait(b)
                        gather(base + grp * nbuf + b + nbuf, b)
            for b in range(nbuf):
                swait(b)
            if with_deg:
                def drain(j, carry):
                    pltpu.make_async_copy(ones_v.at[pl.ds(0, ch)],
                                          deg_sh.at[si_v.at[0]],
                                          dsem).wait()
                    return carry
                lax.fori_loop(0, RP, drain, 0)

        @pl.when(c == 0)
        def _():
            run(t0, gi0, si0)

        @pl.when(c == 1)
        def _():
            run(t1, gi1, si1)

        plsc.subcore_barrier()
        row0 = s * OP
        ooff = 0
        for k, sz in enumerate(OCH):
            b = k % nbuf
            rr = row0 + ooff
            pltpu.sync_copy(accum.at[pl.ds(rr, sz), :],
                            rows[b].at[pl.ds(0, sz), :])
            pltpu.async_copy(rows[b].at[pl.ds(0, sz), :],
                             out.at[c, pl.ds(rr, sz), :], ssem[b])
            ooff += sz
        ooff = 0
        for k, sz in enumerate(OCH):
            b = k % nbuf
            rr = row0 + ooff
            pltpu.make_async_copy(rows[b].at[pl.ds(0, sz), :],
                                  out.at[c, pl.ds(rr, sz), :], ssem[b]).wait()
            ooff += sz

        @pl.when(s == _NSUB - 1)
        def _():
            rr = OP * _NSUB
            pltpu.sync_copy(accum.at[pl.ds(rr, RTAIL), :],
                            r0buf.at[pl.ds(0, RTAIL), :])
            pltpu.sync_copy(r0buf.at[pl.ds(0, RTAIL), :],
                            out.at[c, pl.ds(rr, RTAIL), :])
        if with_deg:
            @pl.when(s < 15)
            def _():
                pltpu.sync_copy(deg_sh.at[pl.ds(s * 640, 640)], dstage_v)

                @pl.when(c == 0)
                def _():
                    pltpu.sync_copy(dstage_v,
                                    deg_out0.at[pl.ds(s * 640, 640)])

                @pl.when(c == 1)
                def _():
                    pltpu.sync_copy(dstage_v,
                                    deg_out1.at[pl.ds(s * 640, 640)])

            @pl.when(s == 15)
            def _():
                pltpu.sync_copy(deg_sh.at[pl.ds(9600, 400)],
                                dstage_v.at[pl.ds(0, 400)])

                @pl.when(c == 0)
                def _():
                    pltpu.sync_copy(dstage_v.at[pl.ds(0, 400)],
                                    deg_out0.at[pl.ds(9600, 400)])

                @pl.when(c == 1)
                def _():
                    pltpu.sync_copy(dstage_v.at[pl.ds(0, 400)],
                                    deg_out1.at[pl.ds(9600, 400)])

    return pl.kernel(body, out_type=out_type, mesh=mesh,
                     scratch_types=scratch)


def _pad_rows(n_edges, ch, iblk):
    unit = ch * _NSUB * iblk
    return -(-n_edges // unit) * _NSUB * iblk


def _pad_idx(idx, rows, ch, scatter):
    pad = rows * ch - idx.shape[0]
    fill = jax.lax.iota(jnp.int32, pad)
    fill = _N + (fill % _NGARB) if scatter else fill % _N
    return jnp.concatenate([idx, fill]).reshape(rows, ch)


def _dot(x, w):
    return jnp.dot(x, w, preferred_element_type=jnp.float32)


@functools.cache
def _tc_layer(relu_out):
    def body(a_ref, R_ref, b_ref, F_ref, ds_ref, dd_ref, Ws_ref, Wn_ref,
             bias_ref, h0_ref, h1_ref):
        inv_s = 1.0 / jnp.maximum(ds_ref[...], 1.0)
        inv_d = 1.0 / jnp.maximum(dd_ref[...], 1.0)
        Ws = Ws_ref[...]
        Wn = Wn_ref[...]
        bias = bias_ref[...]
        h0 = _dot(a_ref[...], Ws) + _dot(R_ref[0] * inv_s, Wn) + bias
        h1 = _dot(b_ref[...], Ws) + _dot(F_ref[0] * inv_d, Wn) + bias
        if relu_out:
            h0 = jnp.maximum(h0, 0.0)
            h1 = jnp.maximum(h1, 0.0)
        h0_ref[...] = h0
        h1_ref[...] = h1

    blk = lambda i: (i, 0)
    return pl.pallas_call(
        body,
        grid=(_N // _BN,),
        in_specs=[
            pl.BlockSpec((_BN, _D), blk),
            pl.BlockSpec((1, _BN, _D), lambda i: (0, i, 0)),
            pl.BlockSpec((_BN, _D), blk),
            pl.BlockSpec((1, _BN, _D), lambda i: (1, i, 0)),
            pl.BlockSpec((_BN, 1), blk),
            pl.BlockSpec((_BN, 1), blk),
            pl.BlockSpec((_D, _D), lambda i: (0, 0)),
            pl.BlockSpec((_D, _D), lambda i: (0, 0)),
            pl.BlockSpec((1, _D), lambda i: (0, 0)),
        ],
        out_specs=[pl.BlockSpec((_BN, _D), blk), pl.BlockSpec((_BN, _D), blk)],
        out_shape=[jax.ShapeDtypeStruct((_N, _D), jnp.float32)] * 2,
    )


@functools.cache
def _tc_final(Dout):
    def body(b_ref, F0_ref, F1_ref, dd_ref, Ws_ref, Wn_ref, bias_ref, o_ref):
        inv_d = 1.0 / jnp.maximum(dd_ref[...], 1.0)
        mean = (F0_ref[0] + F1_ref[0]) * inv_d
        o_ref[...] = (_dot(b_ref[...], Ws_ref[...]) + _dot(mean, Wn_ref[...])
                      + bias_ref[...])

    blk = lambda i: (i, 0)
    return pl.pallas_call(
        body,
        grid=(_N // _BN,),
        in_specs=[
            pl.BlockSpec((_BN, _D), blk),
            pl.BlockSpec((1, _BN, _D), lambda i: (0, i, 0)),
            pl.BlockSpec((1, _BN, _D), lambda i: (1, i, 0)),
            pl.BlockSpec((_BN, 1), blk),
            pl.BlockSpec((_D, Dout), lambda i: (0, 0)),
            pl.BlockSpec((_D, Dout), lambda i: (0, 0)),
            pl.BlockSpec((1, Dout), lambda i: (0, 0)),
        ],
        out_specs=pl.BlockSpec((_BN, Dout), blk),
        out_shape=jax.ShapeDtypeStruct((_N, Dout), jnp.float32),
    )


_CHA, _NBA, _IBA = 88, 3, 24
_CHB, _NBB, _IBB = 88, 3, 24


def kernel(x0, x1, edge_index, W_self_0, W_neigh_0, b_0, W_self_1, W_neigh_1,
           b_1, W_self_2, W_neigh_2, b_2):
    src = edge_index[0]
    dst = edge_index[1]
    E = src.shape[0]
    Dout = W_self_2.shape[1]

    rows_a = _pad_rows(E, _CHA, _IBA)
    h = E // 2
    rows_h = _pad_rows(h, _CHB, _IBB)
    src_g = _pad_idx(src, rows_a, _CHA, False)
    src_s = _pad_idx(src, rows_a, _CHA, True)
    dst_g = _pad_idx(dst, rows_a, _CHA, False)
    dst_s = _pad_idx(dst, rows_a, _CHA, True)

    agg0, deg0, deg1 = _sc_agg(rows_a, True, _CHA, _NBA, _IBA)(
        x1, x0, dst_g, src_s, src_g, dst_s)
    degs = deg0.reshape(_N, 1)
    degd = deg1.reshape(_N, 1)
    a1, b1 = _tc_layer(True)(x0, agg0, x1, agg0, degs, degd,
                             W_self_0, W_neigh_0, b_0.reshape(1, _D))

    agg1 = _sc_agg(rows_a, False, _CHA, _NBA, _IBA)(
        b1, a1, dst_g, src_s, src_g, dst_s)
    a2, b2 = _tc_layer(True)(a1, agg1, b1, agg1, degs, degd,
                             W_self_1, W_neigh_1, b_1.reshape(1, _D))

    agg2 = _sc_agg(rows_h, False, _CHB, _NBB, _IBB)(
        a2, a2,
        _pad_idx(src[:h], rows_h, _CHB, False),
        _pad_idx(dst[:h], rows_h, _CHB, True),
        _pad_idx(src[h:], rows_h, _CHB, False),
        _pad_idx(dst[h:], rows_h, _CHB, True))
    return _tc_final(Dout)(b2, agg2, agg2, degd,
                           W_self_2, W_neigh_2, b_2.reshape(1, Dout))


# speedup vs baseline: 8.3988x; 1.0620x over previous
import functools

import jax
import jax.numpy as jnp
from jax import lax
from jax.experimental import pallas as pl
from jax.experimental.pallas import tpu as pltpu
from jax.experimental.pallas import tpu_sc as plsc

_N = 10000
_D = 128
_NSUB = 16
_NGARB = 64
_NACC = 10112
_BN = 1000


@functools.cache
def _sc_agg(rows_total, with_deg, ch, nbuf, iblk):
    RP = rows_total // _NSUB
    assert RP % iblk == 0 and iblk % nbuf == 0 and iblk % 8 == 0
    NBLK = RP // iblk
    NGRP = iblk // nbuf

    mesh = plsc.VectorSubcoreMesh(core_axis_name="c", subcore_axis_name="s")

    if with_deg:
        out_type = [jax.ShapeDtypeStruct((2, _N, _D), jnp.float32),
                    jax.ShapeDtypeStruct((_N,), jnp.float32),
                    jax.ShapeDtypeStruct((_N,), jnp.float32)]
    else:
        out_type = jax.ShapeDtypeStruct((2, _N, _D), jnp.float32)

    scratch = [
        pltpu.VMEM_SHARED((_NACC, _D), jnp.float32),
        pltpu.VMEM((2 * iblk, ch), jnp.int32),
        pltpu.VMEM((2 * iblk, ch), jnp.int32),
    ]
    scratch += [pltpu.VMEM((ch, _D), jnp.float32) for _ in range(nbuf)]
    scratch += [pltpu.SemaphoreType.DMA]
    scratch += [pltpu.SemaphoreType.DMA for _ in range(nbuf)]
    scratch += [pltpu.SemaphoreType.DMA for _ in range(nbuf)]
    if with_deg:
        scratch += [
            pltpu.VMEM_SHARED((_NACC,), jnp.float32),
            pltpu.VMEM((-(-ch // 16) * 16,), jnp.float32),
            pltpu.VMEM((640,), jnp.float32),
            pltpu.SemaphoreType.DMA,
        ]

    ZP = _NACC // _NSUB
    nz, zr_ = divmod(ZP, ch)
    ZCH = [ch] * nz + ([zr_] if zr_ else [])
    OP = 624
    no, or_ = divmod(OP, ch)
    OCH = [ch] * no + ([or_] if or_ else [])
    RTAIL = _N - OP * _NSUB
    assert all(x % 8 == 0 for x in ZCH + OCH)

    def body(t0, t1, gi0, si0, gi1, si1, out, *rest):
        if with_deg:
            deg_out0, deg_out1, *rest = rest
        accum, gi_v, si_v = rest[:3]
        rows = rest[3:3 + nbuf]
        isem = rest[3 + nbuf]
        gsem = rest[4 + nbuf:4 + 2 * nbuf]
        ssem = rest[4 + 2 * nbuf:4 + 3 * nbuf]
        if with_deg:
            deg_sh, ones_v, dstage_v, dsem = rest[4 + 3 * nbuf:]
        r0buf = rows[0]

        c = lax.axis_index("c")
        s = lax.axis_index("s")
        r0 = s * RP

        def idx_fetch(gi, si, blk):
            half = lax.rem(blk, 2) * iblk
            pltpu.async_copy(gi.at[pl.ds(r0 + blk * iblk, iblk), :],
                             gi_v.at[pl.ds(half, iblk), :], isem)
            pltpu.async_copy(si.at[pl.ds(r0 + blk * iblk, iblk), :],
                             si_v.at[pl.ds(half, iblk), :], isem)

        def idx_wait():
            pltpu.make_async_copy(gi0.at[pl.ds(0, iblk), :],
                                  gi_v.at[pl.ds(0, iblk), :], isem).wait()
            pltpu.make_async_copy(si0.at[pl.ds(0, iblk), :],
                                  si_v.at[pl.ds(0, iblk), :], isem).wait()

        @pl.when(c == 0)
        def _():
            idx_fetch(gi0, si0, 0)

        @pl.when(c == 1)
        def _():
            idx_fetch(gi1, si1, 0)

        def zrow(r, carry):
            for k in range(_D // 16):
                r0buf[r, pl.ds(k * 16, 16)] = jnp.zeros((16,), jnp.float32)
            return carry
        lax.fori_loop(0, ch, zrow, 0)
        zoff = 0
        for sz in ZCH:
            pltpu.async_copy(r0buf.at[pl.ds(0, sz), :],
                             accum.at[pl.ds(s * ZP + zoff, sz), :], ssem[0])
            zoff += sz
        if with_deg:
            def zd(k, carry):
                dstage_v[pl.ds(k * 16, 16)] = jnp.zeros((16,), jnp.float32)
                return carry
            lax.fori_loop(0, 40, zd, 0)

            @pl.when(s < 15)
            def _():
                pltpu.async_copy(dstage_v, deg_sh.at[pl.ds(s * 640, 640)],
                                 dsem)

            @pl.when(s == 15)
            def _():
                pltpu.async_copy(dstage_v.at[pl.ds(0, 512)],
                                 deg_sh.at[pl.ds(9600, 512)], dsem)

            def on(k, carry):
                ones_v[pl.ds(k * 16, 16)] = jnp.full((16,), 1.0, jnp.float32)
                return carry
            lax.fori_loop(0, -(-ch // 16), on, 0)
        zoff = 0
        for sz in ZCH:
            pltpu.make_async_copy(r0buf.at[pl.ds(0, sz), :],
                                  accum.at[pl.ds(s * ZP + zoff, sz), :],
                                  ssem[0]).wait()
            zoff += sz
        if with_deg:
            @pl.when(s < 15)
            def _():
                pltpu.make_async_copy(dstage_v,
                                      deg_sh.at[pl.ds(s * 640, 640)],
                                      dsem).wait()

            @pl.when(s == 15)
            def _():
                pltpu.make_async_copy(dstage_v.at[pl.ds(0, 512)],
                                      deg_sh.at[pl.ds(9600, 512)],
                                      dsem).wait()
        idx_wait()
        plsc.subcore_barrier()

        def run(tab, gi, si):
            def gather(j, b):
                pltpu.async_copy(tab.at[gi_v.at[lax.rem(j, 2 * iblk)]],
                                 rows[b], gsem[b])

            def scatter(j, b):
                jm = lax.rem(j, 2 * iblk)
                pltpu.async_copy(rows[b], accum.at[si_v.at[jm]], ssem[b],
                                 add=True)
                if with_deg:
                    pltpu.async_copy(ones_v.at[pl.ds(0, ch)],
                                     deg_sh.at[si_v.at[jm]], dsem, add=True)

            def gwait(b):
                pltpu.make_async_copy(tab.at[pl.ds(0, ch), :], rows[b],
                                      gsem[b]).wait()

            def swait(b):
                pltpu.make_async_copy(tab.at[pl.ds(0, ch), :], rows[b],
                                      ssem[b]).wait()

            for b in range(nbuf):
                gather(b, b)

            def block(g, carry):
                idx_fetch(gi, si, g + 1)
                base = g * iblk
                for grp in range(NGRP):
                    for b in range(nbuf):
                        gwait(b)
                        scatter(base + grp * nbuf + b, b)
                    if grp == NGRP - 1:
                        idx_wait()
                    for b in range(nbuf):
                        swait(b)
                        gather(base + grp * nbuf + b + nbuf, b)
                return carry
            lax.fori_loop(0, NBLK - 1, block, 0)

            base = (NBLK - 1) * iblk
            for grp in range(NGRP):
                for b in range(nbuf):
                    gwait(b)
                    scatter(base + grp * nbuf + b, b)
                if grp < NGRP - 1:
                    for b in range(nbuf):
                        swait(b)
                        gather(base + grp * nbuf + b + nbuf, b)
            for b in range(nbuf):
                swait(b)
            if with_deg:
                def drain(j, carry):
                    pltpu.make_async_copy(ones_v.at[pl.ds(0, ch)],
                                          deg_sh.at[si_v.at[0]],
                                          dsem).wait()
                    return carry
                lax.fori_loop(0, RP, drain, 0)

        @pl.when(c == 0)
        def _():
            run(t0, gi0, si0)

        @pl.when(c == 1)
        def _():
            run(t1, gi1, si1)

        plsc.subcore_barrier()
        row0 = s * OP
        ooff = 0
        for k, sz in enumerate(OCH):
            b = k % nbuf
            rr = row0 + ooff
            pltpu.sync_copy(accum.at[pl.ds(rr, sz), :],
                            rows[b].at[pl.ds(0, sz), :])
            pltpu.async_copy(rows[b].at[pl.ds(0, sz), :],
                             out.at[c, pl.ds(rr, sz), :], ssem[b])
            ooff += sz
        ooff = 0
        for k, sz in enumerate(OCH):
            b = k % nbuf
            rr = row0 + ooff
            pltpu.make_async_copy(rows[b].at[pl.ds(0, sz), :],
                                  out.at[c, pl.ds(rr, sz), :], ssem[b]).wait()
            ooff += sz

        @pl.when(s == _NSUB - 1)
        def _():
            rr = OP * _NSUB
            pltpu.sync_copy(accum.at[pl.ds(rr, RTAIL), :],
                            r0buf.at[pl.ds(0, RTAIL), :])
            pltpu.sync_copy(r0buf.at[pl.ds(0, RTAIL), :],
                            out.at[c, pl.ds(rr, RTAIL), :])
        if with_deg:
            @pl.when(s < 15)
            def _():
                pltpu.sync_copy(deg_sh.at[pl.ds(s * 640, 640)], dstage_v)

                @pl.when(c == 0)
                def _():
                    pltpu.sync_copy(dstage_v,
                                    deg_out0.at[pl.ds(s * 640, 640)])

                @pl.when(c == 1)
                def _():
                    pltpu.sync_copy(dstage_v,
                                    deg_out1.at[pl.ds(s * 640, 640)])

            @pl.when(s == 15)
            def _():
                pltpu.sync_copy(deg_sh.at[pl.ds(9600, 400)],
                                dstage_v.at[pl.ds(0, 400)])

                @pl.when(c == 0)
                def _():
                    pltpu.sync_copy(dstage_v.at[pl.ds(0, 400)],
                                    deg_out0.at[pl.ds(9600, 400)])

                @pl.when(c == 1)
                def _():
                    pltpu.sync_copy(dstage_v.at[pl.ds(0, 400)],
                                    deg_out1.at[pl.ds(9600, 400)])

    return pl.kernel(body, out_type=out_type, mesh=mesh,
                     scratch_types=scratch)


def _pad_rows(n_edges, ch, iblk):
    unit = ch * _NSUB * iblk
    return -(-n_edges // unit) * _NSUB * iblk


def _pad_idx(idx, rows, ch, scatter):
    pad = rows * ch - idx.shape[0]
    fill = jax.lax.iota(jnp.int32, pad)
    fill = _N + (fill % _NGARB) if scatter else fill % _N
    return jnp.concatenate([idx, fill]).reshape(rows, ch)


def _dot(x, w):
    return jnp.dot(x, w, preferred_element_type=jnp.float32)


@functools.cache
def _tc_layer(relu_out):
    def body(a_ref, R_ref, b_ref, F_ref, ds_ref, dd_ref, Ws_ref, Wn_ref,
             bias_ref, h0_ref, h1_ref):
        inv_s = 1.0 / jnp.maximum(ds_ref[...], 1.0)
        inv_d = 1.0 / jnp.maximum(dd_ref[...], 1.0)
        Ws = Ws_ref[...]
        Wn = Wn_ref[...]
        bias = bias_ref[...]
        h0 = _dot(a_ref[...], Ws) + _dot(R_ref[0] * inv_s, Wn) + bias
        h1 = _dot(b_ref[...], Ws) + _dot(F_ref[0] * inv_d, Wn) + bias
        if relu_out:
            h0 = jnp.maximum(h0, 0.0)
            h1 = jnp.maximum(h1, 0.0)
        h0_ref[...] = h0
        h1_ref[...] = h1

    blk = lambda i: (i, 0)
    return pl.pallas_call(
        body,
        grid=(_N // _BN,),
        in_specs=[
            pl.BlockSpec((_BN, _D), blk),
            pl.BlockSpec((1, _BN, _D), lambda i: (0, i, 0)),
            pl.BlockSpec((_BN, _D), blk),
            pl.BlockSpec((1, _BN, _D), lambda i: (1, i, 0)),
            pl.BlockSpec((_BN, 1), blk),
            pl.BlockSpec((_BN, 1), blk),
            pl.BlockSpec((_D, _D), lambda i: (0, 0)),
            pl.BlockSpec((_D, _D), lambda i: (0, 0)),
            pl.BlockSpec((1, _D), lambda i: (0, 0)),
        ],
        out_specs=[pl.BlockSpec((_BN, _D), blk), pl.BlockSpec((_BN, _D), blk)],
        out_shape=[jax.ShapeDtypeStruct((_N, _D), jnp.float32)] * 2,
    )


@functools.cache
def _tc_final(Dout):
    def body(b_ref, F0_ref, F1_ref, dd_ref, Ws_ref, Wn_ref, bias_ref, o_ref):
        inv_d = 1.0 / jnp.maximum(dd_ref[...], 1.0)
        mean = (F0_ref[0] + F1_ref[0]) * inv_d
        o_ref[...] = (_dot(b_ref[...], Ws_ref[...]) + _dot(mean, Wn_ref[...])
                      + bias_ref[...])

    blk = lambda i: (i, 0)
    return pl.pallas_call(
        body,
        grid=(_N // _BN,),
        in_specs=[
            pl.BlockSpec((_BN, _D), blk),
            pl.BlockSpec((1, _BN, _D), lambda i: (0, i, 0)),
            pl.BlockSpec((1, _BN, _D), lambda i: (1, i, 0)),
            pl.BlockSpec((_BN, 1), blk),
            pl.BlockSpec((_D, Dout), lambda i: (0, 0)),
            pl.BlockSpec((_D, Dout), lambda i: (0, 0)),
            pl.BlockSpec((1, Dout), lambda i: (0, 0)),
        ],
        out_specs=pl.BlockSpec((_BN, Dout), blk),
        out_shape=jax.ShapeDtypeStruct((_N, Dout), jnp.float32),
    )


_CHA, _NBA, _IBA = 72, 4, 16
_CHB, _NBB, _IBB = 72, 4, 16


def kernel(x0, x1, edge_index, W_self_0, W_neigh_0, b_0, W_self_1, W_neigh_1,
           b_1, W_self_2, W_neigh_2, b_2):
    src = edge_index[0]
    dst = edge_index[1]
    E = src.shape[0]
    Dout = W_self_2.shape[1]

    rows_a = _pad_rows(E, _CHA, _IBA)
    h = E // 2
    rows_h = _pad_rows(h, _CHB, _IBB)
    src_g = _pad_idx(src, rows_a, _CHA, False)
    src_s = _pad_idx(src, rows_a, _CHA, True)
    dst_g = _pad_idx(dst, rows_a, _CHA, False)
    dst_s = _pad_idx(dst, rows_a, _CHA, True)

    agg0, deg0, deg1 = _sc_agg(rows_a, True, _CHA, _NBA, _IBA)(
        x1, x0, dst_g, src_s, src_g, dst_s)
    degs = deg0.reshape(_N, 1)
    degd = deg1.reshape(_N, 1)
    a1, b1 = _tc_layer(True)(x0, agg0, x1, agg0, degs, degd,
                             W_self_0, W_neigh_0, b_0.reshape(1, _D))

    agg1 = _sc_agg(rows_a, False, _CHA, _NBA, _IBA)(
        b1, a1, dst_g, src_s, src_g, dst_s)
    a2, b2 = _tc_layer(True)(a1, agg1, b1, agg1, degs, degd,
                             W_self_1, W_neigh_1, b_1.reshape(1, _D))

    agg2 = _sc_agg(rows_h, False, _CHB, _NBB, _IBB)(
        a2, a2,
        _pad_idx(src[:h], rows_h, _CHB, False),
        _pad_idx(dst[:h], rows_h, _CHB, True),
        _pad_idx(src[h:], rows_h, _CHB, False),
        _pad_idx(dst[h:], rows_h, _CHB, True))
    return _tc_final(Dout)(b2, agg2, agg2, degd,
                           W_self_2, W_neigh_2, b_2.reshape(1, Dout))


# speedup vs baseline: 8.4728x; 1.0088x over previous
import functools

import jax
import jax.numpy as jnp
from jax import lax
from jax.experimental import pallas as pl
from jax.experimental.pallas import tpu as pltpu
from jax.experimental.pallas import tpu_sc as plsc

_N = 10000
_D = 128
_NSUB = 16
_NGARB = 64
_NACC = 10112
_BN = 1000


@functools.cache
def _sc_agg(rows_total, with_deg, ch, nbuf, iblk):
    RP = rows_total // _NSUB
    assert RP % iblk == 0 and iblk % nbuf == 0 and iblk % 8 == 0
    NBLK = RP // iblk
    NGRP = iblk // nbuf

    mesh = plsc.VectorSubcoreMesh(core_axis_name="c", subcore_axis_name="s")

    if with_deg:
        out_type = [jax.ShapeDtypeStruct((2, _N, _D), jnp.float32),
                    jax.ShapeDtypeStruct((_N,), jnp.float32),
                    jax.ShapeDtypeStruct((_N,), jnp.float32)]
    else:
        out_type = jax.ShapeDtypeStruct((2, _N, _D), jnp.float32)

    scratch = [
        pltpu.VMEM_SHARED((_NACC, _D), jnp.float32),
        pltpu.VMEM((2 * iblk, ch), jnp.int32),
        pltpu.VMEM((2 * iblk, ch), jnp.int32),
    ]
    scratch += [pltpu.VMEM((ch, _D), jnp.float32) for _ in range(nbuf)]
    scratch += [pltpu.SemaphoreType.DMA]
    scratch += [pltpu.SemaphoreType.DMA for _ in range(nbuf)]
    scratch += [pltpu.SemaphoreType.DMA for _ in range(nbuf)]
    if with_deg:
        scratch += [
            pltpu.VMEM_SHARED((_NACC,), jnp.float32),
            pltpu.VMEM((-(-ch // 16) * 16,), jnp.float32),
            pltpu.VMEM((640,), jnp.float32),
            pltpu.SemaphoreType.DMA,
        ]

    ZP = _NACC // _NSUB
    nz, zr_ = divmod(ZP, ch)
    ZCH = [ch] * nz + ([zr_] if zr_ else [])
    OP = 624
    no, or_ = divmod(OP, ch)
    OCH = [ch] * no + ([or_] if or_ else [])
    RTAIL = _N - OP * _NSUB
    assert all(x % 8 == 0 for x in ZCH + OCH)

    def body(t0, t1, gi0, si0, gi1, si1, out, *rest):
        if with_deg:
            deg_out0, deg_out1, *rest = rest
        accum, gi_v, si_v = rest[:3]
        rows = rest[3:3 + nbuf]
        isem = rest[3 + nbuf]
        gsem = rest[4 + nbuf:4 + 2 * nbuf]
        ssem = rest[4 + 2 * nbuf:4 + 3 * nbuf]
        if with_deg:
            deg_sh, ones_v, dstage_v, dsem = rest[4 + 3 * nbuf:]
        r0buf = rows[0]

        c = lax.axis_index("c")
        s = lax.axis_index("s")
        r0 = s * RP

        def idx_fetch(gi, si, blk):
            half = lax.rem(blk, 2) * iblk
            pltpu.async_copy(gi.at[pl.ds(r0 + blk * iblk, iblk), :],
                             gi_v.at[pl.ds(half, iblk), :], isem)
            pltpu.async_copy(si.at[pl.ds(r0 + blk * iblk, iblk), :],
                             si_v.at[pl.ds(half, iblk), :], isem)

        def idx_wait():
            pltpu.make_async_copy(gi0.at[pl.ds(0, iblk), :],
                                  gi_v.at[pl.ds(0, iblk), :], isem).wait()
            pltpu.make_async_copy(si0.at[pl.ds(0, iblk), :],
                                  si_v.at[pl.ds(0, iblk), :], isem).wait()

        @pl.when(c == 0)
        def _():
            idx_fetch(gi0, si0, 0)

        @pl.when(c == 1)
        def _():
            idx_fetch(gi1, si1, 0)

        def zrow(r, carry):
            for k in range(_D // 16):
                r0buf[r, pl.ds(k * 16, 16)] = jnp.zeros((16,), jnp.float32)
            return carry
        lax.fori_loop(0, ch, zrow, 0)
        zoff = 0
        for sz in ZCH:
            pltpu.async_copy(r0buf.at[pl.ds(0, sz), :],
                             accum.at[pl.ds(s * ZP + zoff, sz), :], ssem[0])
            zoff += sz
        if with_deg:
            def zd(k, carry):
                dstage_v[pl.ds(k * 16, 16)] = jnp.zeros((16,), jnp.float32)
                return carry
            lax.fori_loop(0, 40, zd, 0)

            @pl.when(s < 15)
            def _():
                pltpu.async_copy(dstage_v, deg_sh.at[pl.ds(s * 640, 640)],
                                 dsem)

            @pl.when(s == 15)
            def _():
                pltpu.async_copy(dstage_v.at[pl.ds(0, 512)],
                                 deg_sh.at[pl.ds(9600, 512)], dsem)

            def on(k, carry):
                ones_v[pl.ds(k * 16, 16)] = jnp.full((16,), 1.0, jnp.float32)
                return carry
            lax.fori_loop(0, -(-ch // 16), on, 0)
        zoff = 0
        for sz in ZCH:
            pltpu.make_async_copy(r0buf.at[pl.ds(0, sz), :],
                                  accum.at[pl.ds(s * ZP + zoff, sz), :],
                                  ssem[0]).wait()
            zoff += sz
        if with_deg:
            @pl.when(s < 15)
            def _():
                pltpu.make_async_copy(dstage_v,
                                      deg_sh.at[pl.ds(s * 640, 640)],
                                      dsem).wait()

            @pl.when(s == 15)
            def _():
                pltpu.make_async_copy(dstage_v.at[pl.ds(0, 512)],
                                      deg_sh.at[pl.ds(9600, 512)],
                                      dsem).wait()
        idx_wait()
        plsc.subcore_barrier()

        def run(tab, gi, si):
            def gather(j, b):
                pltpu.async_copy(tab.at[gi_v.at[lax.rem(j, 2 * iblk)]],
                                 rows[b], gsem[b])

            def scatter(j, b):
                jm = lax.rem(j, 2 * iblk)
                pltpu.async_copy(rows[b], accum.at[si_v.at[jm]], ssem[b],
                                 add=True)
                if with_deg:
                    pltpu.async_copy(ones_v.at[pl.ds(0, ch)],
                                     deg_sh.at[si_v.at[jm]], dsem, add=True)

            def gwait(b):
                pltpu.make_async_copy(tab.at[pl.ds(0, ch), :], rows[b],
                                      gsem[b]).wait()

            def swait(b):
                pltpu.make_async_copy(tab.at[pl.ds(0, ch), :], rows[b],
                                      ssem[b]).wait()

            for b in range(nbuf):
                gather(b, b)

            def block(g, carry):
                idx_fetch(gi, si, g + 1)
                base = g * iblk
                for grp in range(NGRP):
                    for b in range(nbuf):
                        gwait(b)
                        scatter(base + grp * nbuf + b, b)
                    if grp == NGRP - 1:
                        idx_wait()
                    for b in range(nbuf):
                        swait(b)
                        gather(base + grp * nbuf + b + nbuf, b)
                return carry
            lax.fori_loop(0, NBLK - 1, block, 0)

            base = (NBLK - 1) * iblk
            for grp in range(NGRP):
                for b in range(nbuf):
                    gwait(b)
                    scatter(base + grp * nbuf + b, b)
                if grp < NGRP - 1:
                    for b in range(nbuf):
                        swait(b)
                        gather(base + grp * nbuf + b + nbuf, b)
            for b in range(nbuf):
                swait(b)
            if with_deg:
                def drain(j, carry):
                    pltpu.make_async_copy(ones_v.at[pl.ds(0, ch)],
                                          deg_sh.at[si_v.at[0]],
                                          dsem).wait()
                    return carry
                lax.fori_loop(0, RP, drain, 0)

        @pl.when(c == 0)
        def _():
            run(t0, gi0, si0)

        @pl.when(c == 1)
        def _():
            run(t1, gi1, si1)

        plsc.subcore_barrier()
        row0 = s * OP
        ooff = 0
        for k, sz in enumerate(OCH):
            b = k % nbuf
            rr = row0 + ooff
            pltpu.sync_copy(accum.at[pl.ds(rr, sz), :],
                            rows[b].at[pl.ds(0, sz), :])
            pltpu.async_copy(rows[b].at[pl.ds(0, sz), :],
                             out.at[c, pl.ds(rr, sz), :], ssem[b])
            ooff += sz
        ooff = 0
        for k, sz in enumerate(OCH):
            b = k % nbuf
            rr = row0 + ooff
            pltpu.make_async_copy(rows[b].at[pl.ds(0, sz), :],
                                  out.at[c, pl.ds(rr, sz), :], ssem[b]).wait()
            ooff += sz

        @pl.when(s == _NSUB - 1)
        def _():
            rr = OP * _NSUB
            pltpu.sync_copy(accum.at[pl.ds(rr, RTAIL), :],
                            r0buf.at[pl.ds(0, RTAIL), :])
            pltpu.sync_copy(r0buf.at[pl.ds(0, RTAIL), :],
                            out.at[c, pl.ds(rr, RTAIL), :])
        if with_deg:
            @pl.when(s < 15)
            def _():
                pltpu.sync_copy(deg_sh.at[pl.ds(s * 640, 640)], dstage_v)

                @pl.when(c == 0)
                def _():
                    pltpu.sync_copy(dstage_v,
                                    deg_out0.at[pl.ds(s * 640, 640)])

                @pl.when(c == 1)
                def _():
                    pltpu.sync_copy(dstage_v,
                                    deg_out1.at[pl.ds(s * 640, 640)])

            @pl.when(s == 15)
            def _():
                pltpu.sync_copy(deg_sh.at[pl.ds(9600, 400)],
                                dstage_v.at[pl.ds(0, 400)])

                @pl.when(c == 0)
                def _():
                    pltpu.sync_copy(dstage_v.at[pl.ds(0, 400)],
                                    deg_out0.at[pl.ds(9600, 400)])

                @pl.when(c == 1)
                def _():
                    pltpu.sync_copy(dstage_v.at[pl.ds(0, 400)],
                                    deg_out1.at[pl.ds(9600, 400)])

    return pl.kernel(body, out_type=out_type, mesh=mesh,
                     scratch_types=scratch)


def _pad_rows(n_edges, ch, iblk):
    unit = ch * _NSUB * iblk
    return -(-n_edges // unit) * _NSUB * iblk


def _pad_idx(idx, rows, ch, scatter):
    pad = rows * ch - idx.shape[0]
    fill = jax.lax.iota(jnp.int32, pad)
    fill = _N + (fill % _NGARB) if scatter else fill % _N
    return jnp.concatenate([idx, fill]).reshape(rows, ch)


def _dot(x, w):
    return jnp.dot(x, w, preferred_element_type=jnp.float32)


@functools.cache
def _tc_layer(relu_out):
    def body(a_ref, R_ref, b_ref, F_ref, ds_ref, dd_ref, Ws_ref, Wn_ref,
             bias_ref, h0_ref, h1_ref):
        inv_s = 1.0 / jnp.maximum(ds_ref[...], 1.0)
        inv_d = 1.0 / jnp.maximum(dd_ref[...], 1.0)
        Ws = Ws_ref[...]
        Wn = Wn_ref[...]
        bias = bias_ref[...]
        h0 = _dot(a_ref[...], Ws) + _dot(R_ref[0] * inv_s, Wn) + bias
        h1 = _dot(b_ref[...], Ws) + _dot(F_ref[0] * inv_d, Wn) + bias
        if relu_out:
            h0 = jnp.maximum(h0, 0.0)
            h1 = jnp.maximum(h1, 0.0)
        h0_ref[...] = h0
        h1_ref[...] = h1

    blk = lambda i: (i, 0)
    return pl.pallas_call(
        body,
        grid=(_N // _BN,),
        in_specs=[
            pl.BlockSpec((_BN, _D), blk),
            pl.BlockSpec((1, _BN, _D), lambda i: (0, i, 0)),
            pl.BlockSpec((_BN, _D), blk),
            pl.BlockSpec((1, _BN, _D), lambda i: (1, i, 0)),
            pl.BlockSpec((_BN, 1), blk),
            pl.BlockSpec((_BN, 1), blk),
            pl.BlockSpec((_D, _D), lambda i: (0, 0)),
            pl.BlockSpec((_D, _D), lambda i: (0, 0)),
            pl.BlockSpec((1, _D), lambda i: (0, 0)),
        ],
        out_specs=[pl.BlockSpec((_BN, _D), blk), pl.BlockSpec((_BN, _D), blk)],
        out_shape=[jax.ShapeDtypeStruct((_N, _D), jnp.float32)] * 2,
    )


@functools.cache
def _tc_final(Dout):
    def body(b_ref, F0_ref, F1_ref, dd_ref, Ws_ref, Wn_ref, bias_ref, o_ref):
        inv_d = 1.0 / jnp.maximum(dd_ref[...], 1.0)
        mean = (F0_ref[0] + F1_ref[0]) * inv_d
        o_ref[...] = (_dot(b_ref[...], Ws_ref[...]) + _dot(mean, Wn_ref[...])
                      + bias_ref[...])

    blk = lambda i: (i, 0)
    return pl.pallas_call(
        body,
        grid=(_N // _BN,),
        in_specs=[
            pl.BlockSpec((_BN, _D), blk),
            pl.BlockSpec((1, _BN, _D), lambda i: (0, i, 0)),
            pl.BlockSpec((1, _BN, _D), lambda i: (1, i, 0)),
            pl.BlockSpec((_BN, 1), blk),
            pl.BlockSpec((_D, Dout), lambda i: (0, 0)),
            pl.BlockSpec((_D, Dout), lambda i: (0, 0)),
            pl.BlockSpec((1, Dout), lambda i: (0, 0)),
        ],
        out_specs=pl.BlockSpec((_BN, Dout), blk),
        out_shape=jax.ShapeDtypeStruct((_N, Dout), jnp.float32),
    )


_CHA, _NBA, _IBA = 64, 4, 16
_CHB, _NBB, _IBB = 64, 4, 16


def kernel(x0, x1, edge_index, W_self_0, W_neigh_0, b_0, W_self_1, W_neigh_1,
           b_1, W_self_2, W_neigh_2, b_2):
    src = edge_index[0]
    dst = edge_index[1]
    E = src.shape[0]
    Dout = W_self_2.shape[1]

    rows_a = _pad_rows(E, _CHA, _IBA)
    h = E // 2
    rows_h = _pad_rows(h, _CHB, _IBB)
    src_g = _pad_idx(src, rows_a, _CHA, False)
    src_s = _pad_idx(src, rows_a, _CHA, True)
    dst_g = _pad_idx(dst, rows_a, _CHA, False)
    dst_s = _pad_idx(dst, rows_a, _CHA, True)

    agg0, deg0, deg1 = _sc_agg(rows_a, True, _CHA, _NBA, _IBA)(
        x1, x0, dst_g, src_s, src_g, dst_s)
    degs = deg0.reshape(_N, 1)
    degd = deg1.reshape(_N, 1)
    a1, b1 = _tc_layer(True)(x0, agg0, x1, agg0, degs, degd,
                             W_self_0, W_neigh_0, b_0.reshape(1, _D))

    agg1 = _sc_agg(rows_a, False, _CHA, _NBA, _IBA)(
        b1, a1, dst_g, src_s, src_g, dst_s)
    a2, b2 = _tc_layer(True)(a1, agg1, b1, agg1, degs, degd,
                             W_self_1, W_neigh_1, b_1.reshape(1, _D))

    agg2 = _sc_agg(rows_h, False, _CHB, _NBB, _IBB)(
        a2, a2,
        _pad_idx(src[:h], rows_h, _CHB, False),
        _pad_idx(dst[:h], rows_h, _CHB, True),
        _pad_idx(src[h:], rows_h, _CHB, False),
        _pad_idx(dst[h:], rows_h, _CHB, True))
    return _tc_final(Dout)(b2, agg2, agg2, degd,
                           W_self_2, W_neigh_2, b_2.reshape(1, Dout))
